# Initial kernel scaffold; baseline (speedup 1.0000x reference)
import functools

import jax
import jax.numpy as jnp
from jax.experimental import pallas as pl
from jax.experimental.pallas import tpu as pltpu

BN_EPS = 1e-5
LEAKY_SLOPE = 0.2
VMEM_LIMIT_BYTES = 32 * 1024 * 1024


def _lrelu(z):
    return jnp.maximum(z, LEAKY_SLOPE * z)


def _stats_kernel(*refs, bn, has_affine):
    if has_affine:
        a_ref, w1_ref, s_ref, t_ref, sum_ref, ssq_ref = refs
    else:
        a_ref, w1_ref, sum_ref, ssq_ref = refs
    j = pl.program_id(1)

    @pl.when(j == 0)
    def _():
        sum_ref[...] = jnp.zeros_like(sum_ref)
        ssq_ref[...] = jnp.zeros_like(ssq_ref)

    w1 = w1_ref[...]
    for i in range(bn):
        a = a_ref[i]
        if has_affine:
            a = _lrelu(a.astype(jnp.float32) * s_ref[...] + t_ref[...])
        y = jnp.dot(w1, a.astype(jnp.bfloat16),
                    preferred_element_type=jnp.float32)
        sum_ref[...] += jnp.sum(y, axis=1, keepdims=True)
        ssq_ref[...] += jnp.sum(y * y, axis=1, keepdims=True)


def _main_kernel(*refs, bn, has_affine):
    if has_affine:
        (a_ref, w1_ref, t1_ref, w2_ref, s_ref, t_ref,
         y2_ref, sum_ref, ssq_ref) = refs
    else:
        a_ref, w1_ref, t1_ref, w2_ref, y2_ref, sum_ref, ssq_ref = refs
    j = pl.program_id(1)

    @pl.when(j == 0)
    def _():
        sum_ref[...] = jnp.zeros_like(sum_ref)
        ssq_ref[...] = jnp.zeros_like(ssq_ref)

    w1 = w1_ref[...]
    w2 = w2_ref[...]
    for i in range(bn):
        a = a_ref[i]
        if has_affine:
            a = _lrelu(a.astype(jnp.float32) * s_ref[...] + t_ref[...])
        y1 = jnp.dot(w1, a.astype(jnp.bfloat16),
                     preferred_element_type=jnp.float32)
        z1 = _lrelu(y1 + t1_ref[...])
        y2 = jnp.dot(w2, z1.astype(jnp.bfloat16),
                     preferred_element_type=jnp.float32)
        y2_ref[i] = y2.astype(y2_ref.dtype)
        sum_ref[...] += jnp.sum(y2, axis=1, keepdims=True)
        ssq_ref[...] += jnp.sum(y2 * y2, axis=1, keepdims=True)


def _final_kernel(y2_ref, s_ref, t_ref, o_ref, *, bn):
    for i in range(bn):
        o_ref[i] = _lrelu(y2_ref[i].astype(jnp.float32) * s_ref[...]
                          + t_ref[...])


def _fold_bn(ch_sum, ch_ssq, gamma, beta, m_real):
    mean = ch_sum / m_real
    var = jnp.maximum(ch_ssq / m_real - mean * mean, 0.0)
    scale = gamma * jax.lax.rsqrt(var + BN_EPS)
    shift = beta - mean * scale
    return scale, shift


def kernel(x,
           w1_0, g1_0, b1_0, w2_0, g2_0, b2_0,
           w1_1, g1_1, b1_1, w2_1, g2_1, b2_1,
           w1_2, g1_2, b1_2, w2_2, g2_2, b2_2):
    params = [((w1_0, g1_0, b1_0), (w2_0, g2_0, b2_0)),
              ((w1_1, g1_1, b1_1), (w2_1, g2_1, b2_1)),
              ((w1_2, g1_2, b1_2), (w2_2, g2_2, b2_2))]

    n, c_in, h, w = x.shape
    hw = h * w
    m_real = n * hw
    a = x.reshape(n, c_in, hw)

    bn = 2
    num_cores = 2 if n >= 2 * bn else 1
    steps = -(-n // bn)
    spc = steps // num_cores
    grid = (num_cores, spc)

    cp_acc = pltpu.CompilerParams(dimension_semantics=("parallel", "arbitrary"),
                                  vmem_limit_bytes=VMEM_LIMIT_BYTES)
    cp_par = pltpu.CompilerParams(dimension_semantics=("parallel", "parallel"),
                                  vmem_limit_bytes=VMEM_LIMIT_BYTES)

    def act_spec(ch):
        return pl.BlockSpec((bn, ch, hw),
                            lambda core, j: (core * spc + j, 0, 0))

    def full_spec(shape):
        nd = len(shape)
        return pl.BlockSpec(tuple(shape), lambda core, j: (0,) * nd)

    def acc_spec(ch):
        return pl.BlockSpec((None, ch, 1), lambda core, j: (core, 0, 0))

    def acc_shape(ch):
        return jax.ShapeDtypeStruct((num_cores, ch, 1), jnp.float32)

    w1b0 = w1_0.astype(jnp.bfloat16)
    c1_0 = w1b0.shape[0]
    sum1, ssq1 = pl.pallas_call(
        functools.partial(_stats_kernel, bn=bn, has_affine=False),
        grid=grid,
        in_specs=[act_spec(c_in), full_spec(w1b0.shape)],
        out_specs=(acc_spec(c1_0), acc_spec(c1_0)),
        out_shape=(acc_shape(c1_0), acc_shape(c1_0)),
        compiler_params=cp_acc,
    )(a, w1b0)
    sum1, ssq1 = sum1.sum(axis=0), ssq1.sum(axis=0)

    nblocks = len(params)
    s2 = t2 = None
    for bi, ((w1, g1, b1), (w2, g2, b2)) in enumerate(params):
        cin = a.shape[1]
        c2 = w2.shape[0]
        has_aff = bi > 0

        s1, t1 = _fold_bn(sum1, ssq1, g1, b1, m_real)
        w1f = (w1 * s1).astype(jnp.bfloat16)
        w2b = w2.astype(jnp.bfloat16)

        ins = [a, w1f, t1, w2b] + ([s2, t2] if has_aff else [])
        in_specs = ([act_spec(cin), full_spec(w1f.shape), full_spec(t1.shape),
                     full_spec(w2b.shape)]
                    + ([full_spec(s2.shape), full_spec(t2.shape)]
                       if has_aff else []))
        y2, sum2, ssq2 = pl.pallas_call(
            functools.partial(_main_kernel, bn=bn, has_affine=has_aff),
            grid=grid,
            in_specs=in_specs,
            out_specs=(act_spec(c2), acc_spec(c2), acc_spec(c2)),
            out_shape=(jax.ShapeDtypeStruct((n, c2, hw), jnp.bfloat16),
                       acc_shape(c2), acc_shape(c2)),
            compiler_params=cp_acc,
        )(*ins)
        sum2, ssq2 = sum2.sum(axis=0), ssq2.sum(axis=0)
        s2, t2 = _fold_bn(sum2, ssq2, g2, b2, m_real)
        a = y2

        if bi + 1 < nblocks:
            wnb = params[bi + 1][0][0].astype(jnp.bfloat16)
            c1n = wnb.shape[0]
            sum1, ssq1 = pl.pallas_call(
                functools.partial(_stats_kernel, bn=bn, has_affine=True),
                grid=grid,
                in_specs=[act_spec(c2), full_spec(wnb.shape),
                          full_spec(s2.shape), full_spec(t2.shape)],
                out_specs=(acc_spec(c1n), acc_spec(c1n)),
                out_shape=(acc_shape(c1n), acc_shape(c1n)),
                compiler_params=cp_acc,
            )(a, wnb, s2, t2)
            sum1, ssq1 = sum1.sum(axis=0), ssq1.sum(axis=0)

    c_out = a.shape[1]
    out = pl.pallas_call(
        functools.partial(_final_kernel, bn=bn),
        grid=grid,
        in_specs=[act_spec(c_out), full_spec(s2.shape), full_spec(t2.shape)],
        out_specs=act_spec(c_out),
        out_shape=jax.ShapeDtypeStruct((n, c_out, hw), jnp.float32),
        compiler_params=cp_par,
    )(a, s2, t2)
    return out.reshape(n, c_out, h, w)


# speedup vs baseline: 1.1448x; 1.1448x over previous
import functools

import jax
import jax.numpy as jnp
from jax.experimental import pallas as pl
from jax.experimental.pallas import tpu as pltpu

BN_EPS = 1e-5
LEAKY_SLOPE = 0.2
VMEM_LIMIT_BYTES = 32 * 1024 * 1024
_DOT_DT = jnp.float32
_MID_DT = jnp.float32


def _lrelu(z):
    return jnp.maximum(z, LEAKY_SLOPE * z)


def _stats_kernel(*refs, bn, has_affine):
    if has_affine:
        a_ref, w1_ref, s_ref, t_ref, sum_ref, ssq_ref = refs
    else:
        a_ref, w1_ref, sum_ref, ssq_ref = refs
    j = pl.program_id(1)

    @pl.when(j == 0)
    def _():
        sum_ref[...] = jnp.zeros_like(sum_ref)
        ssq_ref[...] = jnp.zeros_like(ssq_ref)

    w1 = w1_ref[...]
    for i in range(bn):
        a = a_ref[i]
        if has_affine:
            a = _lrelu(a.astype(jnp.float32) * s_ref[...] + t_ref[...])
        y = jnp.dot(w1, a.astype(_DOT_DT),
                    preferred_element_type=jnp.float32)
        sum_ref[...] += jnp.sum(y, axis=1, keepdims=True)
        ssq_ref[...] += jnp.sum(y * y, axis=1, keepdims=True)


def _main_kernel(*refs, bn, has_affine):
    if has_affine:
        (a_ref, w1_ref, t1_ref, w2_ref, s_ref, t_ref,
         y2_ref, sum_ref, ssq_ref) = refs
    else:
        a_ref, w1_ref, t1_ref, w2_ref, y2_ref, sum_ref, ssq_ref = refs
    j = pl.program_id(1)

    @pl.when(j == 0)
    def _():
        sum_ref[...] = jnp.zeros_like(sum_ref)
        ssq_ref[...] = jnp.zeros_like(ssq_ref)

    w1 = w1_ref[...]
    w2 = w2_ref[...]
    for i in range(bn):
        a = a_ref[i]
        if has_affine:
            a = _lrelu(a.astype(jnp.float32) * s_ref[...] + t_ref[...])
        y1 = jnp.dot(w1, a.astype(_DOT_DT),
                     preferred_element_type=jnp.float32)
        z1 = _lrelu(y1 + t1_ref[...])
        y2 = jnp.dot(w2, z1.astype(_DOT_DT),
                     preferred_element_type=jnp.float32)
        y2_ref[i] = y2.astype(y2_ref.dtype)
        sum_ref[...] += jnp.sum(y2, axis=1, keepdims=True)
        ssq_ref[...] += jnp.sum(y2 * y2, axis=1, keepdims=True)


def _final_kernel(y2_ref, s_ref, t_ref, o_ref, *, bn):
    for i in range(bn):
        o_ref[i] = _lrelu(y2_ref[i].astype(jnp.float32) * s_ref[...]
                          + t_ref[...])


def _fold_bn(ch_sum, ch_ssq, gamma, beta, m_real):
    mean = ch_sum / m_real
    var = jnp.maximum(ch_ssq / m_real - mean * mean, 0.0)
    scale = gamma * jax.lax.rsqrt(var + BN_EPS)
    shift = beta - mean * scale
    return scale, shift


def kernel(x,
           w1_0, g1_0, b1_0, w2_0, g2_0, b2_0,
           w1_1, g1_1, b1_1, w2_1, g2_1, b2_1,
           w1_2, g1_2, b1_2, w2_2, g2_2, b2_2):
    params = [((w1_0, g1_0, b1_0), (w2_0, g2_0, b2_0)),
              ((w1_1, g1_1, b1_1), (w2_1, g2_1, b2_1)),
              ((w1_2, g1_2, b1_2), (w2_2, g2_2, b2_2))]

    n, c_in, h, w = x.shape
    hw = h * w
    m_real = n * hw
    a = x.reshape(n, c_in, hw)

    bn = 2
    num_cores = 2 if n >= 2 * bn else 1
    steps = -(-n // bn)
    spc = steps // num_cores
    grid = (num_cores, spc)

    cp_acc = pltpu.CompilerParams(dimension_semantics=("parallel", "arbitrary"),
                                  vmem_limit_bytes=VMEM_LIMIT_BYTES)
    cp_par = pltpu.CompilerParams(dimension_semantics=("parallel", "parallel"),
                                  vmem_limit_bytes=VMEM_LIMIT_BYTES)

    def act_spec(ch):
        return pl.BlockSpec((bn, ch, hw),
                            lambda core, j: (core * spc + j, 0, 0))

    def full_spec(shape):
        nd = len(shape)
        return pl.BlockSpec(tuple(shape), lambda core, j: (0,) * nd)

    def acc_spec(ch):
        return pl.BlockSpec((None, ch, 1), lambda core, j: (core, 0, 0))

    def acc_shape(ch):
        return jax.ShapeDtypeStruct((num_cores, ch, 1), jnp.float32)

    w1b0 = w1_0.astype(_DOT_DT)
    c1_0 = w1b0.shape[0]
    sum1, ssq1 = pl.pallas_call(
        functools.partial(_stats_kernel, bn=bn, has_affine=False),
        grid=grid,
        in_specs=[act_spec(c_in), full_spec(w1b0.shape)],
        out_specs=(acc_spec(c1_0), acc_spec(c1_0)),
        out_shape=(acc_shape(c1_0), acc_shape(c1_0)),
        compiler_params=cp_acc,
    )(a, w1b0)
    sum1, ssq1 = sum1.sum(axis=0), ssq1.sum(axis=0)

    nblocks = len(params)
    s2 = t2 = None
    for bi, ((w1, g1, b1), (w2, g2, b2)) in enumerate(params):
        cin = a.shape[1]
        c2 = w2.shape[0]
        has_aff = bi > 0

        s1, t1 = _fold_bn(sum1, ssq1, g1, b1, m_real)
        w1f = (w1 * s1).astype(_DOT_DT)
        w2b = w2.astype(_DOT_DT)

        ins = [a, w1f, t1, w2b] + ([s2, t2] if has_aff else [])
        in_specs = ([act_spec(cin), full_spec(w1f.shape), full_spec(t1.shape),
                     full_spec(w2b.shape)]
                    + ([full_spec(s2.shape), full_spec(t2.shape)]
                       if has_aff else []))
        y2, sum2, ssq2 = pl.pallas_call(
            functools.partial(_main_kernel, bn=bn, has_affine=has_aff),
            grid=grid,
            in_specs=in_specs,
            out_specs=(act_spec(c2), acc_spec(c2), acc_spec(c2)),
            out_shape=(jax.ShapeDtypeStruct((n, c2, hw), _MID_DT),
                       acc_shape(c2), acc_shape(c2)),
            compiler_params=cp_acc,
        )(*ins)
        sum2, ssq2 = sum2.sum(axis=0), ssq2.sum(axis=0)
        s2, t2 = _fold_bn(sum2, ssq2, g2, b2, m_real)
        a = y2

        if bi + 1 < nblocks:
            wnb = params[bi + 1][0][0].astype(_DOT_DT)
            c1n = wnb.shape[0]
            sum1, ssq1 = pl.pallas_call(
                functools.partial(_stats_kernel, bn=bn, has_affine=True),
                grid=grid,
                in_specs=[act_spec(c2), full_spec(wnb.shape),
                          full_spec(s2.shape), full_spec(t2.shape)],
                out_specs=(acc_spec(c1n), acc_spec(c1n)),
                out_shape=(acc_shape(c1n), acc_shape(c1n)),
                compiler_params=cp_acc,
            )(a, wnb, s2, t2)
            sum1, ssq1 = sum1.sum(axis=0), ssq1.sum(axis=0)

    c_out = a.shape[1]
    out = pl.pallas_call(
        functools.partial(_final_kernel, bn=bn),
        grid=grid,
        in_specs=[act_spec(c_out), full_spec(s2.shape), full_spec(t2.shape)],
        out_specs=act_spec(c_out),
        out_shape=jax.ShapeDtypeStruct((n, c_out, hw), jnp.float32),
        compiler_params=cp_par,
    )(a, s2, t2)
    return out.reshape(n, c_out, h, w)
